# Initial kernel scaffold; baseline (speedup 1.0000x reference)
import jax
import jax.numpy as jnp
from jax.experimental import pallas as pl


def kernel(logits, targets):
    raise NotImplementedError("write your pallas kernel here")


# speedup vs baseline: 2.5585x; 2.5585x over previous
import functools

import jax
import jax.numpy as jnp
from jax import lax
from jax.experimental import pallas as pl
from jax.experimental.pallas import tpu as pltpu

NUM_CLASSES = 1000
EPS = 0.1
BATCH = 16384
A = EPS / (NUM_CLASSES - 1)
B_COEF = 1.0 - EPS - A

BR = 512


def _loss_body(x_ref, t_ref, out_ref):
    i = pl.program_id(0)
    x = x_ref[...]
    t = t_ref[0, 0, :]
    m = jnp.max(x, axis=1, keepdims=True)
    s = jnp.sum(jnp.exp(x - m), axis=1)
    lse = jnp.log(s) + m[:, 0]
    row_sum = jnp.sum(x, axis=1)
    col = lax.broadcasted_iota(jnp.int32, x.shape, 1)
    tgt = jnp.sum(jnp.where(col == t[:, None], x, 0.0), axis=1)
    part = jnp.sum(lse - A * row_sum - B_COEF * tgt) * (1.0 / BATCH)

    @pl.when(i == 0)
    def _():
        out_ref[...] = jnp.zeros((1, 1), jnp.float32)

    out_ref[...] += jnp.reshape(part, (1, 1))


@jax.jit
def kernel(logits, targets):
    n_blocks = BATCH // BR
    t3 = targets.astype(jnp.int32).reshape(n_blocks, 1, BR)
    out = pl.pallas_call(
        _loss_body,
        grid=(n_blocks,),
        in_specs=[
            pl.BlockSpec((BR, NUM_CLASSES), lambda i: (i, 0)),
            pl.BlockSpec((1, 1, BR), lambda i: (i, 0, 0)),
        ],
        out_specs=pl.BlockSpec((1, 1), lambda i: (0, 0)),
        out_shape=jax.ShapeDtypeStruct((1, 1), jnp.float32),
    )(logits, t3)
    return out[0, 0]


# speedup vs baseline: 2.8226x; 1.1032x over previous
import functools

import jax
import jax.numpy as jnp
from jax import lax
from jax.experimental import pallas as pl
from jax.experimental.pallas import tpu as pltpu

NUM_CLASSES = 1000
EPS = 0.1
BATCH = 16384
A = EPS / (NUM_CLASSES - 1)
B_COEF = 1.0 - EPS - A

BR = 1024


def _loss_body(x_ref, t_ref, out_ref):
    i = pl.program_id(0)
    x = x_ref[...]
    t = t_ref[0, 0, :]
    m = jnp.max(x, axis=1, keepdims=True)
    s = jnp.sum(jnp.exp(x - m), axis=1)
    lse = jnp.log(s) + m[:, 0]
    row_sum = jnp.sum(x, axis=1)
    col = lax.broadcasted_iota(jnp.int32, x.shape, 1)
    tgt = jnp.sum(jnp.where(col == t[:, None], x, 0.0), axis=1)
    part = jnp.sum(lse - A * row_sum - B_COEF * tgt) * (1.0 / BATCH)

    @pl.when(i == 0)
    def _():
        out_ref[...] = jnp.zeros((1, 1), jnp.float32)

    out_ref[...] += jnp.reshape(part, (1, 1))


@jax.jit
def kernel(logits, targets):
    n_blocks = BATCH // BR
    t3 = targets.astype(jnp.int32).reshape(n_blocks, 1, BR)
    out = pl.pallas_call(
        _loss_body,
        grid=(n_blocks,),
        in_specs=[
            pl.BlockSpec((BR, NUM_CLASSES), lambda i: (i, 0)),
            pl.BlockSpec((1, 1, BR), lambda i: (i, 0, 0)),
        ],
        out_specs=pl.BlockSpec((1, 1), lambda i: (0, 0)),
        out_shape=jax.ShapeDtypeStruct((1, 1), jnp.float32),
    )(logits, t3)
    return out[0, 0]


# speedup vs baseline: 2.8951x; 1.0257x over previous
import functools

import jax
import jax.numpy as jnp
from jax import lax
from jax.experimental import pallas as pl
from jax.experimental.pallas import tpu as pltpu

NUM_CLASSES = 1000
EPS = 0.1
BATCH = 16384
A = EPS / (NUM_CLASSES - 1)
B_COEF = 1.0 - EPS - A

BR = 2048


def _loss_body(x_ref, t_ref, out_ref):
    i = pl.program_id(0)
    x = x_ref[...]
    t = t_ref[0, 0, :]
    m = jnp.max(x, axis=1, keepdims=True)
    s = jnp.sum(jnp.exp(x - m), axis=1)
    lse = jnp.log(s) + m[:, 0]
    row_sum = jnp.sum(x, axis=1)
    col = lax.broadcasted_iota(jnp.int32, x.shape, 1)
    tgt = jnp.sum(jnp.where(col == t[:, None], x, 0.0), axis=1)
    part = jnp.sum(lse - A * row_sum - B_COEF * tgt) * (1.0 / BATCH)

    @pl.when(i == 0)
    def _():
        out_ref[...] = jnp.zeros((1, 1), jnp.float32)

    out_ref[...] += jnp.reshape(part, (1, 1))


@jax.jit
def kernel(logits, targets):
    n_blocks = BATCH // BR
    t3 = targets.astype(jnp.int32).reshape(n_blocks, 1, BR)
    out = pl.pallas_call(
        _loss_body,
        grid=(n_blocks,),
        in_specs=[
            pl.BlockSpec((BR, NUM_CLASSES), lambda i: (i, 0)),
            pl.BlockSpec((1, 1, BR), lambda i: (i, 0, 0)),
        ],
        out_specs=pl.BlockSpec((1, 1), lambda i: (0, 0)),
        out_shape=jax.ShapeDtypeStruct((1, 1), jnp.float32),
    )(logits, t3)
    return out[0, 0]
